# baseline (speedup 1.0000x reference)
import jax
import jax.numpy as jnp
from jax.experimental import pallas as pl
from jax.experimental.pallas import tpu as pltpu
from jax.experimental.pallas import tpu_sc as plsc
from jax import lax
import functools

E = 8
K = 2
D = 1024
F = 2048
T = 2048
B = 256
NBLK = T * K // B + E
NP = NBLK * B
CH = 256
GB = 1024
CB = 1024


def _routing_kernel(logits_ref, pos_ref, pw_ref, meta_ref):
    l = logits_ref[...]
    iotaE = jax.lax.broadcasted_iota(jnp.int32, (T, E), 1)
    m1 = jnp.max(l, axis=1, keepdims=True)
    i1 = jnp.min(jnp.where(l == m1, iotaE, E), axis=1, keepdims=True)
    lm = jnp.where(iotaE == i1, -jnp.inf, l)
    m2 = jnp.max(lm, axis=1, keepdims=True)
    i2 = jnp.min(jnp.where(lm == m2, iotaE, E), axis=1, keepdims=True)
    e2 = jnp.exp(m2 - m1)
    w1 = 1.0 / (1.0 + e2)
    w2 = e2 / (1.0 + e2)

    oh1 = (iotaE == i1)
    oh2 = (iotaE == i2)
    s = jnp.where(oh1, 1.0, 0.0) + jnp.where(oh2, 1.0, 0.0)

    r = jax.lax.broadcasted_iota(jnp.int32, (CH, CH), 0)
    c = jax.lax.broadcasted_iota(jnp.int32, (CH, CH), 1)
    tri = jnp.where(c < r, 1.0, 0.0)
    chunks = []
    carry = jnp.zeros((1, E), jnp.float32)
    for ci in range(T // CH):
        blk = jax.lax.slice(s, (ci * CH, 0), ((ci + 1) * CH, E))
        within = jax.lax.dot_general(tri, blk, (((1,), (0,)), ((), ())),
                                     preferred_element_type=jnp.float32)
        chunks.append(within + carry)
        carry = carry + jnp.sum(blk, axis=0, keepdims=True)
    C = jnp.concatenate(chunks, axis=0)

    cnt_row = carry
    pc_row = jnp.floor((cnt_row + (B - 1)) / B) * B
    r8 = jax.lax.broadcasted_iota(jnp.int32, (E, E), 0)
    c8 = jax.lax.broadcasted_iota(jnp.int32, (E, E), 1)
    u8 = jnp.where(r8 < c8, 1.0, 0.0)
    off_row = jax.lax.dot_general(pc_row, u8, (((1,), (0,)), ((), ())),
                                  preferred_element_type=jnp.float32)
    total = jnp.sum(pc_row)

    dest = off_row + C
    pos1 = jnp.sum(jnp.where(oh1, dest, 0.0), axis=1, keepdims=True)
    pos2 = jnp.sum(jnp.where(oh2, dest, 0.0), axis=1, keepdims=True)
    pos_ref[...] = jnp.concatenate([pos1, pos2], axis=1).astype(jnp.int32)
    pw_ref[...] = jnp.concatenate([w1, w2], axis=1)

    l8 = jnp.where(c8 < r8, 1.0, 0.0)
    ones_col = jnp.zeros((T, 1), jnp.float32) + 1.0
    cnt_col = jax.lax.dot_general(s, ones_col, (((0,), (0,)), ((), ())),
                                  preferred_element_type=jnp.float32)
    pc_col = jnp.floor((cnt_col + (B - 1)) / B) * B
    off_col = jax.lax.dot_general(l8, pc_col, (((1,), (0,)), ((), ())),
                                  preferred_element_type=jnp.float32)
    bB = (jax.lax.broadcasted_iota(jnp.int32, (1, NBLK), 1) * B).astype(jnp.float32)
    V = jnp.where(off_col <= bB, 1.0, 0.0)
    be = jnp.sum(V, axis=0, keepdims=True) - 1.0
    be = jnp.clip(be, 0.0, E - 1)
    act = jnp.where(bB < total, 1.0, 0.0)
    meta_ref[...] = jnp.concatenate([be, act], axis=0).astype(jnp.int32)


def _gather_kernel(pos_ref, pw_ref, hs_ref, xs_ref, rw_ref):
    b = pl.program_id(0)
    pos = pos_ref[...]
    pw = pw_ref[...]
    lk = jax.lax.broadcasted_iota(jnp.int32, (T, K), 1)
    pos0 = jnp.sum(jnp.where(lk == 0, pos, 0), axis=1, keepdims=True)
    pos1 = jnp.sum(jnp.where(lk == 1, pos, 0), axis=1, keepdims=True)
    w0 = jnp.sum(jnp.where(lk == 0, pw, 0.0), axis=1, keepdims=True)
    w1 = jnp.sum(jnp.where(lk == 1, pw, 0.0), axis=1, keepdims=True)
    rowidx = b * GB + jax.lax.broadcasted_iota(jnp.int32, (1, GB), 1)
    m = (jnp.where(pos0 == rowidx, 1.0, 0.0)
         + jnp.where(pos1 == rowidx, 1.0, 0.0))
    mw = (jnp.where(pos0 == rowidx, w0, 0.0)
          + jnp.where(pos1 == rowidx, w1, 0.0))
    ones = jnp.zeros((T, 1), jnp.float32) + 1.0
    xs_ref[...] = jax.lax.dot_general(m, hs_ref[...], (((0,), (0,)), ((), ())),
                                      preferred_element_type=jnp.float32)
    rw_ref[...] = jax.lax.dot_general(mw, ones, (((0,), (0,)), ((), ())),
                                      preferred_element_type=jnp.float32)


def _mlp_kernel(meta_ref, xs_ref, rw_ref, wg_ref, wu_ref, w2_ref, y_ref):
    b = pl.program_id(0)
    act = meta_ref[1, b]

    @pl.when(act == 1)
    def _():
        x = xs_ref[...]
        gate = jax.lax.dot_general(x, wg_ref[0], (((1,), (1,)), ((), ())),
                                   preferred_element_type=jnp.float32)
        up = jax.lax.dot_general(x, wu_ref[0], (((1,), (1,)), ((), ())),
                                 preferred_element_type=jnp.float32)
        h = (gate * jax.lax.logistic(gate)) * up
        y = jax.lax.dot_general(h, w2_ref[0], (((1,), (1,)), ((), ())),
                                preferred_element_type=jnp.float32)
        y_ref[...] = y * rw_ref[...]

    @pl.when(act == 0)
    def _():
        y_ref[...] = jnp.zeros((B, D), jnp.float32)


NWORK = 32
TPW = T // NWORK
CK = 16


def _sc_combine_kernel(p0_hbm, p1_hbm, yw_hbm, out_hbm,
                       idx0_v, idx1_v, r0_v, r1_v, sem):
    wid = lax.axis_index("s") * 2 + lax.axis_index("c")

    def body(c, carry):
        base = wid * TPW + c * CK
        pltpu.sync_copy(p0_hbm.at[pl.ds(base, CK)], idx0_v)
        pltpu.sync_copy(p1_hbm.at[pl.ds(base, CK)], idx1_v)
        pltpu.async_copy(yw_hbm.at[idx0_v], r0_v, sem).wait()
        pltpu.async_copy(yw_hbm.at[idx1_v], r1_v, sem).wait()
        for i in range(CK):
            for j in range(D // 16):
                sl = pl.ds(j * 16, 16)
                r0_v[i, sl] = r0_v[i, sl] + r1_v[i, sl]
        pltpu.sync_copy(r0_v, out_hbm.at[pl.ds(base, CK)])
        return carry

    jax.lax.fori_loop(0, TPW // CK, body, 0)


def kernel(hidden_states, router_logits, w13_weight, w2_weight):
    pos, pw, meta = pl.pallas_call(
        _routing_kernel,
        out_shape=(
            jax.ShapeDtypeStruct((T, K), jnp.int32),
            jax.ShapeDtypeStruct((T, K), jnp.float32),
            jax.ShapeDtypeStruct((2, NBLK), jnp.int32),
        ),
    )(router_logits.astype(jnp.float32))

    xs, rw = pl.pallas_call(
        _gather_kernel,
        grid=(NP // GB,),
        in_specs=[
            pl.BlockSpec((T, K), lambda b: (0, 0)),
            pl.BlockSpec((T, K), lambda b: (0, 0)),
            pl.BlockSpec((T, D), lambda b: (0, 0)),
        ],
        out_specs=(
            pl.BlockSpec((GB, D), lambda b: (b, 0)),
            pl.BlockSpec((GB, 1), lambda b: (b, 0)),
        ),
        out_shape=(
            jax.ShapeDtypeStruct((NP, D), jnp.float32),
            jax.ShapeDtypeStruct((NP, 1), jnp.float32),
        ),
        compiler_params=pltpu.CompilerParams(
            dimension_semantics=("arbitrary",),
        ),
    )(pos, pw, hidden_states)

    y = pl.pallas_call(
        _mlp_kernel,
        grid_spec=pltpu.PrefetchScalarGridSpec(
            num_scalar_prefetch=1,
            grid=(NBLK,),
            in_specs=[
                pl.BlockSpec((B, D), lambda b, m: (b, 0)),
                pl.BlockSpec((B, 1), lambda b, m: (b, 0)),
                pl.BlockSpec((1, F, D), lambda b, m: (m[0, b], 0, 0)),
                pl.BlockSpec((1, F, D), lambda b, m: (m[0, b], 1, 0)),
                pl.BlockSpec((1, D, F), lambda b, m: (m[0, b], 0, 0)),
            ],
            out_specs=pl.BlockSpec((B, D), lambda b, m: (b, 0)),
        ),
        out_shape=jax.ShapeDtypeStruct((NP, D), jnp.float32),
        compiler_params=pltpu.CompilerParams(
            dimension_semantics=("arbitrary",),
            vmem_limit_bytes=100 * 1024 * 1024,
        ),
    )(meta, xs, rw, w13_weight, w13_weight, w2_weight)

    p0 = jax.lax.slice(pos, (0, 0), (T, 1)).reshape(T)
    p1 = jax.lax.slice(pos, (0, 1), (T, 2)).reshape(T)
    sc_combine = functools.partial(
        pl.kernel,
        out_type=jax.ShapeDtypeStruct((T, D), jnp.float32),
        mesh=plsc.VectorSubcoreMesh(core_axis_name="c", subcore_axis_name="s"),
        scratch_types=[
            pltpu.VMEM((CK,), jnp.int32),
            pltpu.VMEM((CK,), jnp.int32),
            pltpu.VMEM((CK, D), jnp.float32),
            pltpu.VMEM((CK, D), jnp.float32),
            pltpu.SemaphoreType.DMA,
        ],
    )(_sc_combine_kernel)
    return sc_combine(p0, p1, y)


# speedup vs baseline: 1.0030x; 1.0030x over previous
import jax
import jax.numpy as jnp
from jax.experimental import pallas as pl
from jax.experimental.pallas import tpu as pltpu
from jax.experimental.pallas import tpu_sc as plsc
from jax import lax
import functools

E = 8
K = 2
D = 1024
F = 2048
T = 2048
B = 256
NBLK = T * K // B + E
NP = NBLK * B
CH = 256
GB = 1024
CB = 1024


def _routing_kernel(logits_ref, pos_ref, pw_ref, meta_ref):
    l = logits_ref[...]
    iotaE = jax.lax.broadcasted_iota(jnp.int32, (T, E), 1)
    m1 = jnp.max(l, axis=1, keepdims=True)
    i1 = jnp.min(jnp.where(l == m1, iotaE, E), axis=1, keepdims=True)
    lm = jnp.where(iotaE == i1, -jnp.inf, l)
    m2 = jnp.max(lm, axis=1, keepdims=True)
    i2 = jnp.min(jnp.where(lm == m2, iotaE, E), axis=1, keepdims=True)
    e2 = jnp.exp(m2 - m1)
    w1 = 1.0 / (1.0 + e2)
    w2 = e2 / (1.0 + e2)

    oh1 = (iotaE == i1)
    oh2 = (iotaE == i2)
    s = jnp.where(oh1, 1.0, 0.0) + jnp.where(oh2, 1.0, 0.0)

    r = jax.lax.broadcasted_iota(jnp.int32, (CH, CH), 0)
    c = jax.lax.broadcasted_iota(jnp.int32, (CH, CH), 1)
    tri = jnp.where(c < r, 1.0, 0.0)
    chunks = []
    carry = jnp.zeros((1, E), jnp.float32)
    for ci in range(T // CH):
        blk = jax.lax.slice(s, (ci * CH, 0), ((ci + 1) * CH, E))
        within = jax.lax.dot_general(tri, blk, (((1,), (0,)), ((), ())),
                                     preferred_element_type=jnp.float32)
        chunks.append(within + carry)
        carry = carry + jnp.sum(blk, axis=0, keepdims=True)
    C = jnp.concatenate(chunks, axis=0)

    cnt_row = carry
    pc_row = jnp.floor((cnt_row + (B - 1)) / B) * B
    r8 = jax.lax.broadcasted_iota(jnp.int32, (E, E), 0)
    c8 = jax.lax.broadcasted_iota(jnp.int32, (E, E), 1)
    u8 = jnp.where(r8 < c8, 1.0, 0.0)
    off_row = jax.lax.dot_general(pc_row, u8, (((1,), (0,)), ((), ())),
                                  preferred_element_type=jnp.float32)
    total = jnp.sum(pc_row)

    dest = off_row + C
    pos1 = jnp.sum(jnp.where(oh1, dest, 0.0), axis=1, keepdims=True)
    pos2 = jnp.sum(jnp.where(oh2, dest, 0.0), axis=1, keepdims=True)
    pos_ref[...] = jnp.concatenate([pos1, pos2], axis=1).astype(jnp.int32)
    pw_ref[...] = jnp.concatenate([w1, w2], axis=1)

    l8 = jnp.where(c8 < r8, 1.0, 0.0)
    ones_col = jnp.zeros((T, 1), jnp.float32) + 1.0
    cnt_col = jax.lax.dot_general(s, ones_col, (((0,), (0,)), ((), ())),
                                  preferred_element_type=jnp.float32)
    pc_col = jnp.floor((cnt_col + (B - 1)) / B) * B
    off_col = jax.lax.dot_general(l8, pc_col, (((1,), (0,)), ((), ())),
                                  preferred_element_type=jnp.float32)
    bB = (jax.lax.broadcasted_iota(jnp.int32, (1, NBLK), 1) * B).astype(jnp.float32)
    V = jnp.where(off_col <= bB, 1.0, 0.0)
    be = jnp.sum(V, axis=0, keepdims=True) - 1.0
    be = jnp.clip(be, 0.0, E - 1)
    act = jnp.where(bB < total, 1.0, 0.0)
    meta_ref[...] = jnp.concatenate([be, act], axis=0).astype(jnp.int32)


def _gather_kernel(pos_ref, pw_ref, hs_ref, xs_ref, rw_ref):
    b = pl.program_id(0)
    pos = pos_ref[...]
    pw = pw_ref[...]
    lk = jax.lax.broadcasted_iota(jnp.int32, (T, K), 1)
    pos0 = jnp.sum(jnp.where(lk == 0, pos, 0), axis=1, keepdims=True)
    pos1 = jnp.sum(jnp.where(lk == 1, pos, 0), axis=1, keepdims=True)
    w0 = jnp.sum(jnp.where(lk == 0, pw, 0.0), axis=1, keepdims=True)
    w1 = jnp.sum(jnp.where(lk == 1, pw, 0.0), axis=1, keepdims=True)
    rowidx = b * GB + jax.lax.broadcasted_iota(jnp.int32, (1, GB), 1)
    m = (jnp.where(pos0 == rowidx, 1.0, 0.0)
         + jnp.where(pos1 == rowidx, 1.0, 0.0))
    mw = (jnp.where(pos0 == rowidx, w0, 0.0)
          + jnp.where(pos1 == rowidx, w1, 0.0))
    ones = jnp.zeros((T, 1), jnp.float32) + 1.0
    xs_ref[...] = jax.lax.dot_general(m, hs_ref[...], (((0,), (0,)), ((), ())),
                                      preferred_element_type=jnp.float32)
    rw_ref[...] = jax.lax.dot_general(mw, ones, (((0,), (0,)), ((), ())),
                                      preferred_element_type=jnp.float32)


def _mlp_kernel(meta_ref, xs_ref, rw_ref, wg_ref, wu_ref, w2_ref, y_ref):
    b = pl.program_id(0)
    act = meta_ref[1, b]

    @pl.when(act == 1)
    def _():
        x = xs_ref[...]
        gate = jax.lax.dot_general(x, wg_ref[0], (((1,), (1,)), ((), ())),
                                   preferred_element_type=jnp.float32)
        up = jax.lax.dot_general(x, wu_ref[0], (((1,), (1,)), ((), ())),
                                 preferred_element_type=jnp.float32)
        h = (gate * jax.lax.logistic(gate)) * up
        y = jax.lax.dot_general(h, w2_ref[0], (((1,), (1,)), ((), ())),
                                preferred_element_type=jnp.float32)
        y_ref[...] = y * rw_ref[...]

    @pl.when(act == 0)
    def _():
        y_ref[...] = jnp.zeros((B, D), jnp.float32)


NWORK = 32
TPW = T // NWORK
CK = 32


def _sc_combine_kernel(p0_hbm, p1_hbm, yw_hbm, out_hbm,
                       idx0_v, idx1_v, r0_v, r1_v, sem):
    wid = lax.axis_index("s") * 2 + lax.axis_index("c")

    def body(c, carry):
        base = wid * TPW + c * CK
        pltpu.sync_copy(p0_hbm.at[pl.ds(base, CK)], idx0_v)
        pltpu.sync_copy(p1_hbm.at[pl.ds(base, CK)], idx1_v)
        cp0 = pltpu.async_copy(yw_hbm.at[idx0_v], r0_v, sem)
        cp1 = pltpu.async_copy(yw_hbm.at[idx1_v], r1_v, sem)
        cp0.wait()
        cp1.wait()
        for i in range(CK):
            for j in range(D // 16):
                sl = pl.ds(j * 16, 16)
                r0_v[i, sl] = r0_v[i, sl] + r1_v[i, sl]
        pltpu.sync_copy(r0_v, out_hbm.at[pl.ds(base, CK)])
        return carry

    jax.lax.fori_loop(0, TPW // CK, body, 0)


def kernel(hidden_states, router_logits, w13_weight, w2_weight):
    pos, pw, meta = pl.pallas_call(
        _routing_kernel,
        out_shape=(
            jax.ShapeDtypeStruct((T, K), jnp.int32),
            jax.ShapeDtypeStruct((T, K), jnp.float32),
            jax.ShapeDtypeStruct((2, NBLK), jnp.int32),
        ),
    )(router_logits.astype(jnp.float32))

    xs, rw = pl.pallas_call(
        _gather_kernel,
        grid=(NP // GB,),
        in_specs=[
            pl.BlockSpec((T, K), lambda b: (0, 0)),
            pl.BlockSpec((T, K), lambda b: (0, 0)),
            pl.BlockSpec((T, D), lambda b: (0, 0)),
        ],
        out_specs=(
            pl.BlockSpec((GB, D), lambda b: (b, 0)),
            pl.BlockSpec((GB, 1), lambda b: (b, 0)),
        ),
        out_shape=(
            jax.ShapeDtypeStruct((NP, D), jnp.float32),
            jax.ShapeDtypeStruct((NP, 1), jnp.float32),
        ),
        compiler_params=pltpu.CompilerParams(
            dimension_semantics=("arbitrary",),
        ),
    )(pos, pw, hidden_states)

    y = pl.pallas_call(
        _mlp_kernel,
        grid_spec=pltpu.PrefetchScalarGridSpec(
            num_scalar_prefetch=1,
            grid=(NBLK,),
            in_specs=[
                pl.BlockSpec((B, D), lambda b, m: (b, 0)),
                pl.BlockSpec((B, 1), lambda b, m: (b, 0)),
                pl.BlockSpec((1, F, D), lambda b, m: (m[0, b], 0, 0)),
                pl.BlockSpec((1, F, D), lambda b, m: (m[0, b], 1, 0)),
                pl.BlockSpec((1, D, F), lambda b, m: (m[0, b], 0, 0)),
            ],
            out_specs=pl.BlockSpec((B, D), lambda b, m: (b, 0)),
        ),
        out_shape=jax.ShapeDtypeStruct((NP, D), jnp.float32),
        compiler_params=pltpu.CompilerParams(
            dimension_semantics=("arbitrary",),
            vmem_limit_bytes=100 * 1024 * 1024,
        ),
    )(meta, xs, rw, w13_weight, w13_weight, w2_weight)

    p0 = jax.lax.slice(pos, (0, 0), (T, 1)).reshape(T)
    p1 = jax.lax.slice(pos, (0, 1), (T, 2)).reshape(T)
    sc_combine = functools.partial(
        pl.kernel,
        out_type=jax.ShapeDtypeStruct((T, D), jnp.float32),
        mesh=plsc.VectorSubcoreMesh(core_axis_name="c", subcore_axis_name="s"),
        scratch_types=[
            pltpu.VMEM((CK,), jnp.int32),
            pltpu.VMEM((CK,), jnp.int32),
            pltpu.VMEM((CK, D), jnp.float32),
            pltpu.VMEM((CK, D), jnp.float32),
            pltpu.SemaphoreType.DMA,
        ],
    )(_sc_combine_kernel)
    return sc_combine(p0, p1, y)


# speedup vs baseline: 1.2108x; 1.2072x over previous
import jax
import jax.numpy as jnp
from jax.experimental import pallas as pl
from jax.experimental.pallas import tpu as pltpu

E = 8
K = 2
D = 1024
F = 2048
T = 2048
B = 256
NBLK = T * K // B + E
NP = NBLK * B
CH = 256
GB = 1024
CB = 1024


def _routing_kernel(logits_ref, pos_ref, pw_ref, meta_ref):
    l = logits_ref[...]
    iotaE = jax.lax.broadcasted_iota(jnp.int32, (T, E), 1)
    m1 = jnp.max(l, axis=1, keepdims=True)
    i1 = jnp.min(jnp.where(l == m1, iotaE, E), axis=1, keepdims=True)
    lm = jnp.where(iotaE == i1, -jnp.inf, l)
    m2 = jnp.max(lm, axis=1, keepdims=True)
    i2 = jnp.min(jnp.where(lm == m2, iotaE, E), axis=1, keepdims=True)
    e2 = jnp.exp(m2 - m1)
    w1 = 1.0 / (1.0 + e2)
    w2 = e2 / (1.0 + e2)

    oh1 = (iotaE == i1)
    oh2 = (iotaE == i2)
    s = jnp.where(oh1, 1.0, 0.0) + jnp.where(oh2, 1.0, 0.0)

    r = jax.lax.broadcasted_iota(jnp.int32, (CH, CH), 0)
    c = jax.lax.broadcasted_iota(jnp.int32, (CH, CH), 1)
    tri = jnp.where(c < r, 1.0, 0.0)
    chunks = []
    carry = jnp.zeros((1, E), jnp.float32)
    for ci in range(T // CH):
        blk = jax.lax.slice(s, (ci * CH, 0), ((ci + 1) * CH, E))
        within = jax.lax.dot_general(tri, blk, (((1,), (0,)), ((), ())),
                                     preferred_element_type=jnp.float32)
        chunks.append(within + carry)
        carry = carry + jnp.sum(blk, axis=0, keepdims=True)
    C = jnp.concatenate(chunks, axis=0)

    cnt_row = carry
    pc_row = jnp.floor((cnt_row + (B - 1)) / B) * B
    r8 = jax.lax.broadcasted_iota(jnp.int32, (E, E), 0)
    c8 = jax.lax.broadcasted_iota(jnp.int32, (E, E), 1)
    u8 = jnp.where(r8 < c8, 1.0, 0.0)
    off_row = jax.lax.dot_general(pc_row, u8, (((1,), (0,)), ((), ())),
                                  preferred_element_type=jnp.float32)
    total = jnp.sum(pc_row)

    dest = off_row + C
    pos1 = jnp.sum(jnp.where(oh1, dest, 0.0), axis=1, keepdims=True)
    pos2 = jnp.sum(jnp.where(oh2, dest, 0.0), axis=1, keepdims=True)
    pos_ref[...] = jnp.concatenate([pos1, pos2], axis=1).astype(jnp.int32)
    pw_ref[...] = jnp.concatenate([w1, w2], axis=1)

    l8 = jnp.where(c8 < r8, 1.0, 0.0)
    ones_col = jnp.zeros((T, 1), jnp.float32) + 1.0
    cnt_col = jax.lax.dot_general(s, ones_col, (((0,), (0,)), ((), ())),
                                  preferred_element_type=jnp.float32)
    pc_col = jnp.floor((cnt_col + (B - 1)) / B) * B
    off_col = jax.lax.dot_general(l8, pc_col, (((1,), (0,)), ((), ())),
                                  preferred_element_type=jnp.float32)
    bB = (jax.lax.broadcasted_iota(jnp.int32, (1, NBLK), 1) * B).astype(jnp.float32)
    V = jnp.where(off_col <= bB, 1.0, 0.0)
    be = jnp.sum(V, axis=0, keepdims=True) - 1.0
    be = jnp.clip(be, 0.0, E - 1)
    act = jnp.where(bB < total, 1.0, 0.0)
    meta_ref[...] = jnp.concatenate([be, act], axis=0).astype(jnp.int32)


def _gather_kernel(pos_ref, hs_ref, xs_ref):
    b = pl.program_id(0)
    pos = pos_ref[...]
    lk = jax.lax.broadcasted_iota(jnp.int32, (T, K), 1)
    pos0 = jnp.sum(jnp.where(lk == 0, pos, 0), axis=1, keepdims=True)
    pos1 = jnp.sum(jnp.where(lk == 1, pos, 0), axis=1, keepdims=True)
    rowidx = b * GB + jax.lax.broadcasted_iota(jnp.int32, (1, GB), 1)
    m = (jnp.where(pos0 == rowidx, 1.0, 0.0)
         + jnp.where(pos1 == rowidx, 1.0, 0.0))
    xs_ref[...] = jax.lax.dot_general(m, hs_ref[...], (((0,), (0,)), ((), ())),
                                      preferred_element_type=jnp.float32)


def _mlp_kernel(meta_ref, xs_ref, wg_ref, wu_ref, w2_ref, y_ref):
    b = pl.program_id(0)
    act = meta_ref[1, b]

    @pl.when(act == 1)
    def _():
        x = xs_ref[...]
        gate = jax.lax.dot_general(x, wg_ref[0], (((1,), (1,)), ((), ())),
                                   preferred_element_type=jnp.float32)
        up = jax.lax.dot_general(x, wu_ref[0], (((1,), (1,)), ((), ())),
                                 preferred_element_type=jnp.float32)
        h = (gate * jax.lax.logistic(gate)) * up
        y = jax.lax.dot_general(h, w2_ref[0], (((1,), (1,)), ((), ())),
                                preferred_element_type=jnp.float32)
        y_ref[...] = y.astype(jnp.bfloat16)

    @pl.when(act == 0)
    def _():
        y_ref[...] = jnp.zeros((B, D), jnp.bfloat16)


def _combine_kernel(pos_ref, pw_ref, y_ref, out_ref):
    pos = pos_ref[...]
    pw = pw_ref[...]
    lk = jax.lax.broadcasted_iota(jnp.int32, (CB, K), 1)
    p0 = jnp.sum(jnp.where(lk == 0, pos, 0), axis=1, keepdims=True)
    p1 = jnp.sum(jnp.where(lk == 1, pos, 0), axis=1, keepdims=True)
    w0 = jnp.sum(jnp.where(lk == 0, pw, 0.0), axis=1, keepdims=True)
    w1 = jnp.sum(jnp.where(lk == 1, pw, 0.0), axis=1, keepdims=True)
    col = jax.lax.broadcasted_iota(jnp.int32, (1, NP), 1)
    g = (jnp.where(p0 == col, w0, 0.0)
         + jnp.where(p1 == col, w1, 0.0)).astype(jnp.bfloat16)
    out_ref[...] = jax.lax.dot_general(g, y_ref[...], (((1,), (0,)), ((), ())),
                                       preferred_element_type=jnp.float32)


def kernel(hidden_states, router_logits, w13_weight, w2_weight):
    pos, pw, meta = pl.pallas_call(
        _routing_kernel,
        out_shape=(
            jax.ShapeDtypeStruct((T, K), jnp.int32),
            jax.ShapeDtypeStruct((T, K), jnp.float32),
            jax.ShapeDtypeStruct((2, NBLK), jnp.int32),
        ),
    )(router_logits.astype(jnp.float32))

    xs = pl.pallas_call(
        _gather_kernel,
        grid=(NP // GB,),
        in_specs=[
            pl.BlockSpec((T, K), lambda b: (0, 0)),
            pl.BlockSpec((T, D), lambda b: (0, 0)),
        ],
        out_specs=pl.BlockSpec((GB, D), lambda b: (b, 0)),
        out_shape=jax.ShapeDtypeStruct((NP, D), jnp.float32),
        compiler_params=pltpu.CompilerParams(
            dimension_semantics=("arbitrary",),
        ),
    )(pos, hidden_states)

    y = pl.pallas_call(
        _mlp_kernel,
        grid_spec=pltpu.PrefetchScalarGridSpec(
            num_scalar_prefetch=1,
            grid=(NBLK,),
            in_specs=[
                pl.BlockSpec((B, D), lambda b, m: (b, 0)),
                pl.BlockSpec((1, F, D), lambda b, m: (m[0, b], 0, 0)),
                pl.BlockSpec((1, F, D), lambda b, m: (m[0, b], 1, 0)),
                pl.BlockSpec((1, D, F), lambda b, m: (m[0, b], 0, 0)),
            ],
            out_specs=pl.BlockSpec((B, D), lambda b, m: (b, 0)),
        ),
        out_shape=jax.ShapeDtypeStruct((NP, D), jnp.bfloat16),
        compiler_params=pltpu.CompilerParams(
            dimension_semantics=("arbitrary",),
            vmem_limit_bytes=100 * 1024 * 1024,
        ),
    )(meta, xs, w13_weight, w13_weight, w2_weight)

    out = pl.pallas_call(
        _combine_kernel,
        grid=(T // CB,),
        in_specs=[
            pl.BlockSpec((CB, K), lambda t: (t, 0)),
            pl.BlockSpec((CB, K), lambda t: (t, 0)),
            pl.BlockSpec((NP, D), lambda t: (0, 0)),
        ],
        out_specs=pl.BlockSpec((CB, D), lambda t: (t, 0)),
        out_shape=jax.ShapeDtypeStruct((T, D), jnp.float32),
        compiler_params=pltpu.CompilerParams(
            dimension_semantics=("arbitrary",),
        ),
    )(pos, pw, y)
    return out
